# baseline (speedup 1.0000x reference)
import functools

import jax
import jax.numpy as jnp
from jax import lax
from jax.experimental import pallas as pl
from jax.experimental.pallas import tpu as pltpu
from jax.experimental.pallas import tpu_sc as plsc

E = 320000
D = 128
A = 16
NC = 2
NS = 16
NW = NC * NS
EPW = E // NW
CH = 40
NCHUNK = EPW // CH
DEPTH = 5
AHEAD = 3
NGRP = NCHUNK // DEPTH

_mesh = plsc.VectorSubcoreMesh(core_axis_name="c", subcore_axis_name="s")


@functools.partial(
    pl.kernel,
    out_type=jax.ShapeDtypeStruct((E, 2 * D + A), jnp.float32),
    mesh=_mesh,
    scratch_types=[
        pltpu.VMEM((NCHUNK, CH), jnp.int32),
        pltpu.VMEM((NCHUNK, CH), jnp.int32),
        [pltpu.VMEM((CH, D), jnp.float32)] * DEPTH,
        [pltpu.VMEM((CH, D), jnp.float32)] * DEPTH,
        [pltpu.VMEM((CH, A), jnp.float32)] * DEPTH,
        [pltpu.SemaphoreType.DMA] * DEPTH,
        [pltpu.SemaphoreType.DMA] * DEPTH,
    ],
    compiler_params=pltpu.CompilerParams(use_tc_tiling_on_sc=False),
)
def _edge_concat(x_hbm, ei_hbm, ea_hbm, out_hbm,
                 sidx, didx, sbufs, dbufs, abufs, rsems, wsems):
    wid = lax.axis_index("s") * NC + lax.axis_index("c")
    base = wid * EPW

    pltpu.sync_copy(ei_hbm.at[0, wid], sidx)
    pltpu.sync_copy(ei_hbm.at[1, wid], didx)

    def issue_reads(j, s):
        gbase = base + j * CH
        pltpu.async_copy(x_hbm.at[sidx.at[j]], sbufs[s], rsems[s])
        pltpu.async_copy(x_hbm.at[didx.at[j]], dbufs[s], rsems[s])
        pltpu.async_copy(ea_hbm.at[pl.ds(gbase, CH)], abufs[s], rsems[s])

    def wait_reads(s):
        pltpu.make_async_copy(x_hbm.at[sidx.at[0]], sbufs[s], rsems[s]).wait()
        pltpu.make_async_copy(x_hbm.at[didx.at[0]], dbufs[s], rsems[s]).wait()
        pltpu.make_async_copy(ea_hbm.at[pl.ds(base, CH)], abufs[s],
                              rsems[s]).wait()

    def issue_writes(j, s):
        gbase = base + j * CH
        pltpu.async_copy(sbufs[s], out_hbm.at[pl.ds(gbase, CH), pl.ds(0, D)],
                         wsems[s])
        pltpu.async_copy(dbufs[s], out_hbm.at[pl.ds(gbase, CH), pl.ds(D, D)],
                         wsems[s])
        pltpu.async_copy(abufs[s],
                         out_hbm.at[pl.ds(gbase, CH), pl.ds(2 * D, A)],
                         wsems[s])

    def wait_writes(s):
        pltpu.make_async_copy(sbufs[s], out_hbm.at[pl.ds(base, CH), pl.ds(0, D)],
                              wsems[s]).wait()
        pltpu.make_async_copy(dbufs[s], out_hbm.at[pl.ds(base, CH), pl.ds(D, D)],
                              wsems[s]).wait()
        pltpu.make_async_copy(abufs[s],
                              out_hbm.at[pl.ds(base, CH), pl.ds(2 * D, A)],
                              wsems[s]).wait()

    for j in range(AHEAD):
        issue_reads(j, j)

    def grp(q, carry):
        j0 = DEPTH * q
        for b in range(DEPTH):
            j = j0 + b
            t = (b + AHEAD) % DEPTH

            @pl.when(j >= DEPTH - AHEAD)
            def _():
                wait_writes(t)

            @pl.when(j < NCHUNK - AHEAD)
            def _():
                issue_reads(j + AHEAD, t)

            wait_reads(b)
            issue_writes(j, b)
        return carry

    lax.fori_loop(0, NGRP, grp, 0)

    for b in range(DEPTH - AHEAD):
        wait_writes((NCHUNK - (DEPTH - AHEAD) + b) % DEPTH)


def kernel(x, edge_index, edge_attr):
    ei = edge_index.astype(jnp.int32).reshape(2, NW, NCHUNK, CH)
    return _edge_concat(x, ei, edge_attr)


# speedup vs baseline: 1.4417x; 1.4417x over previous
import functools

import jax
import jax.numpy as jnp
from jax import lax
from jax.experimental import pallas as pl
from jax.experimental.pallas import tpu as pltpu
from jax.experimental.pallas import tpu_sc as plsc

E = 320000
D = 128
A = 16
NC = 2
NS = 16
NW = NC * NS
EPW = E // NW
CH = 40
NCHUNK = EPW // CH
DEPTH = 5
AHEAD = 3
NGRP = NCHUNK // DEPTH

_mesh = plsc.VectorSubcoreMesh(core_axis_name="c", subcore_axis_name="s")


@functools.partial(
    pl.kernel,
    out_type=jax.ShapeDtypeStruct((E, 2 * D + A), jnp.float32),
    mesh=_mesh,
    scratch_types=[
        pltpu.VMEM((NCHUNK, 2 * CH), jnp.int32),
        [pltpu.VMEM((CH, D), jnp.float32)] * DEPTH,
        [pltpu.VMEM((CH, D), jnp.float32)] * DEPTH,
        [pltpu.VMEM((CH, A), jnp.float32)] * DEPTH,
        [pltpu.SemaphoreType.DMA] * DEPTH,
        [pltpu.SemaphoreType.DMA] * DEPTH,
    ],
)
def _edge_concat(x_hbm, ei_hbm, ea_hbm, out_hbm,
                 idx, sbufs, dbufs, abufs, rsems, wsems):
    wid = lax.axis_index("s") * NC + lax.axis_index("c")
    base = wid * EPW

    pltpu.sync_copy(ei_hbm.at[wid], idx)

    def issue_reads(j, s):
        gbase = base + j * CH
        pltpu.async_copy(x_hbm.at[idx.at[j, pl.ds(0, CH)]], sbufs[s], rsems[s])
        pltpu.async_copy(x_hbm.at[idx.at[j, pl.ds(CH, CH)]], dbufs[s], rsems[s])
        pltpu.async_copy(ea_hbm.at[pl.ds(gbase, CH)], abufs[s], rsems[s])

    def wait_reads(s):
        pltpu.make_async_copy(x_hbm.at[idx.at[0, pl.ds(0, CH)]], sbufs[s], rsems[s]).wait()
        pltpu.make_async_copy(x_hbm.at[idx.at[0, pl.ds(CH, CH)]], dbufs[s], rsems[s]).wait()
        pltpu.make_async_copy(ea_hbm.at[pl.ds(base, CH)], abufs[s],
                              rsems[s]).wait()

    def issue_writes(j, s):
        gbase = base + j * CH
        pltpu.async_copy(sbufs[s], out_hbm.at[pl.ds(gbase, CH), pl.ds(0, D)],
                         wsems[s])
        pltpu.async_copy(dbufs[s], out_hbm.at[pl.ds(gbase, CH), pl.ds(D, D)],
                         wsems[s])
        pltpu.async_copy(abufs[s],
                         out_hbm.at[pl.ds(gbase, CH), pl.ds(2 * D, A)],
                         wsems[s])

    def wait_writes(s):
        pltpu.make_async_copy(sbufs[s], out_hbm.at[pl.ds(base, CH), pl.ds(0, D)],
                              wsems[s]).wait()
        pltpu.make_async_copy(dbufs[s], out_hbm.at[pl.ds(base, CH), pl.ds(D, D)],
                              wsems[s]).wait()
        pltpu.make_async_copy(abufs[s],
                              out_hbm.at[pl.ds(base, CH), pl.ds(2 * D, A)],
                              wsems[s]).wait()

    for j in range(AHEAD):
        issue_reads(j, j)

    def grp(q, carry):
        j0 = DEPTH * q
        for b in range(DEPTH):
            j = j0 + b
            t = (b + AHEAD) % DEPTH

            @pl.when(j >= DEPTH - AHEAD)
            def _():
                wait_writes(t)

            @pl.when(j < NCHUNK - AHEAD)
            def _():
                issue_reads(j + AHEAD, t)

            wait_reads(b)
            issue_writes(j, b)
        return carry

    lax.fori_loop(0, NGRP, grp, 0)

    for b in range(DEPTH - AHEAD):
        wait_writes((NCHUNK - (DEPTH - AHEAD) + b) % DEPTH)


def kernel(x, edge_index, edge_attr):
    ei = edge_index.astype(jnp.int32).reshape(2, NW, NCHUNK, CH)
    ei = jnp.concatenate([ei[0], ei[1]], axis=-1)
    return _edge_concat(x, ei, edge_attr)
